# baseline (speedup 1.0000x reference)
import functools

import jax
import jax.numpy as jnp
from jax import lax
from jax.experimental import pallas as pl
from jax.experimental.pallas import tpu as pltpu
from jax.experimental.pallas import tpu_sc as plsc

VOCAB = 1_000_000
DIM = 64
NB = 4096
NS = 200
BATCH = NB * NS

NUM_CORES = 2
NUM_SUBCORES = 16
NUM_WORKERS = NUM_CORES * NUM_SUBCORES
ROWS_PER_W = NB // NUM_WORKERS
PER_WORKER = ROWS_PER_W * NS


def _emb_body(idx_hbm, table_hbm, out_hbm, idx_v, jbuf, pbuf, gbuf, obuf, *sems):
    gsems, osems = sems[:2], sems[2:]
    wid = lax.axis_index("s") * NUM_CORES + lax.axis_index("c")
    base = wid * PER_WORKER
    row0 = wid * ROWS_PER_W
    pltpu.sync_copy(idx_hbm.at[pl.ds(base, PER_WORKER)], idx_v)

    lanes = lax.iota(jnp.int32, 16)

    def build_idx(s, b):
        for q in range(8):
            pos = (q * 16 + lanes) * NS + s
            v = plsc.load_gather(idx_v, [pos])
            jbuf[b, pl.ds(q * 16, 16)] = lax.shift_right_logical(v, 1)
            pbuf[b, pl.ds(q * 16, 16)] = lax.bitwise_and(v, 1)

    def gather(s, b):
        return pltpu.make_async_copy(
            table_hbm.at[jbuf.at[b]], gbuf.at[b], gsems[b]
        )

    def outcopy(s, b):
        return pltpu.make_async_copy(
            obuf.at[b],
            out_hbm.at[s, :, pl.ds(row0, ROWS_PER_W)],
            osems[b],
        )

    def pack(b):
        for q in range(8):
            rv = q * 16 + lanes
            colbase = pbuf[b, pl.ds(q * 16, 16)] * DIM

            def dbody(d, _):
                val = plsc.load_gather(gbuf.at[b], [rv, colbase + d])
                obuf[b, d, pl.ds(q * 16, 16)] = val
                return 0

            lax.fori_loop(0, DIM, dbody, 0)

    build_idx(0, 0)
    gather(0, 0).start()

    def turn(g, _):
        for b in range(2):
            s = 2 * g + b

            @pl.when(s + 1 < NS)
            def _():
                build_idx(s + 1, 1 - b)
                gather(s + 1, 1 - b).start()

            gather(s, b).wait()

            @pl.when(s >= 2)
            def _():
                outcopy(s - 2, b).wait()

            pack(b)
            outcopy(s, b).start()
        return 0

    lax.fori_loop(0, NS // 2, turn, 0)
    outcopy(NS - 2, 0).wait()
    outcopy(NS - 1, 1).wait()


@jax.jit
def _embed_fused(idx_flat, table_pairs):
    mesh = plsc.VectorSubcoreMesh(core_axis_name="c", subcore_axis_name="s")
    f = functools.partial(
        pl.kernel,
        mesh=mesh,
        out_type=jax.ShapeDtypeStruct((NS, DIM, NB), jnp.float32),
        scratch_types=[
            pltpu.VMEM((PER_WORKER,), jnp.int32),
            pltpu.VMEM((2, ROWS_PER_W), jnp.int32),
            pltpu.VMEM((2, ROWS_PER_W), jnp.int32),
            pltpu.VMEM((2, ROWS_PER_W, 128), jnp.float32),
            pltpu.VMEM((2, DIM, ROWS_PER_W), jnp.float32),
        ]
        + [pltpu.SemaphoreType.DMA] * 4,
        compiler_params=pltpu.CompilerParams(
            use_tc_tiling_on_sc=True, needs_layout_passes=False
        ),
    )(_emb_body)
    return f(idx_flat, table_pairs)


def kernel(token_ids, embeddings):
    b, s = token_ids.shape
    idx_flat = token_ids.reshape(b * s).astype(jnp.int32)
    table_pairs = embeddings.reshape(VOCAB // 2, 2 * DIM)
    out = _embed_fused(idx_flat, table_pairs)
    return jnp.transpose(out, (2, 0, 1))
